# baseline (speedup 1.0000x reference)
import functools

import jax
import jax.numpy as jnp
from jax import lax
from jax.experimental import pallas as pl
from jax.experimental.pallas import tpu as pltpu
from jax.experimental.pallas import tpu_sc as plsc

D = 128
NC = 2
NS = 16
NW = NC * NS
CH = 64
NTOT = 8
K = 6


@functools.partial(jax.jit, static_argnums=(0,))
def _gather(nch, idx, table):
    B = NW * nch * CH
    mesh = plsc.VectorSubcoreMesh(core_axis_name="c", subcore_axis_name="s")

    @functools.partial(
        pl.kernel,
        mesh=mesh,
        out_type=jax.ShapeDtypeStruct((B, D), jnp.float32),
        scratch_types=[
            pltpu.VMEM((nch, CH), jnp.int32),
            pltpu.VMEM((NTOT, CH, D), jnp.float32),
            pltpu.SemaphoreType.DMA((NTOT,)),
            pltpu.SemaphoreType.DMA((NTOT,)),
        ],
    )
    def k(table_hbm, idx_hbm, out_hbm, idx_v, rows_v, gsem, osem):
        wid = lax.axis_index("s") * NC + lax.axis_index("c")
        pltpu.sync_copy(idx_hbm.at[wid], idx_v)
        base = wid * (nch * CH)

        def start_g(c):
            b = c % NTOT
            pltpu.async_copy(table_hbm.at[idx_v.at[c]], rows_v.at[b], gsem.at[b])

        def wait_g(c):
            b = c % NTOT
            pltpu.make_async_copy(
                table_hbm.at[idx_v.at[c]], rows_v.at[b], gsem.at[b]
            ).wait()

        def start_w(c):
            b = c % NTOT
            dst = out_hbm.at[pl.ds(base + c * CH, CH)]
            pltpu.async_copy(rows_v.at[b], dst, osem.at[b])

        def wait_w(c):
            b = c % NTOT
            dst = out_hbm.at[pl.ds(base + c * CH, CH)]
            pltpu.make_async_copy(rows_v.at[b], dst, osem.at[b]).wait()

        for g in range(K):
            start_g(g)

        def body_early(g, carry):
            start_g(g + K)
            wait_g(g)
            start_w(g)
            return carry

        lax.fori_loop(0, NTOT - K, body_early, 0)

        def body_main(g, carry):
            wait_w(g + K - NTOT)
            start_g(g + K)
            wait_g(g)
            start_w(g)
            return carry

        lax.fori_loop(NTOT - K, nch - K, body_main, 0)

        def body_tail(g, carry):
            wait_g(g)
            start_w(g)
            return carry

        lax.fori_loop(nch - K, nch, body_tail, 0)

        def body_drain(g, carry):
            wait_w(g)
            return carry

        lax.fori_loop(nch - NTOT, nch, body_drain, 0)

    return k(table, idx)


def kernel(nodes_list, id2emb):
    batch, hist = nodes_list.shape
    B = batch * hist
    assert B % (NW * CH) == 0
    nch = B // (NW * CH)
    idx = nodes_list.astype(jnp.int32).reshape(NW, nch, CH)
    out = _gather(nch, idx, id2emb)
    return out.reshape(batch, hist, D)


# speedup vs baseline: 1.0014x; 1.0014x over previous
import functools

import jax
import jax.numpy as jnp
from jax import lax
from jax.experimental import pallas as pl
from jax.experimental.pallas import tpu as pltpu
from jax.experimental.pallas import tpu_sc as plsc

D = 128
NC = 2
NS = 16
NW = NC * NS
CH = 128
NTOT = 6
K = 4


@functools.partial(jax.jit, static_argnums=(0,))
def _gather(nch, idx, table):
    B = NW * nch * CH
    mesh = plsc.VectorSubcoreMesh(core_axis_name="c", subcore_axis_name="s")

    @functools.partial(
        pl.kernel,
        mesh=mesh,
        out_type=jax.ShapeDtypeStruct((B, D), jnp.float32),
        scratch_types=[
            pltpu.VMEM((nch, CH), jnp.int32),
            pltpu.VMEM((NTOT, CH, D), jnp.float32),
            pltpu.SemaphoreType.DMA((NTOT,)),
            pltpu.SemaphoreType.DMA((NTOT,)),
        ],
    )
    def k(table_hbm, idx_hbm, out_hbm, idx_v, rows_v, gsem, osem):
        wid = lax.axis_index("s") * NC + lax.axis_index("c")
        pltpu.sync_copy(idx_hbm.at[wid], idx_v)
        base = wid * (nch * CH)

        def start_g(c):
            b = c % NTOT
            pltpu.async_copy(table_hbm.at[idx_v.at[c]], rows_v.at[b], gsem.at[b])

        def wait_g(c):
            b = c % NTOT
            pltpu.make_async_copy(
                table_hbm.at[idx_v.at[c]], rows_v.at[b], gsem.at[b]
            ).wait()

        def start_w(c):
            b = c % NTOT
            dst = out_hbm.at[pl.ds(base + c * CH, CH)]
            pltpu.async_copy(rows_v.at[b], dst, osem.at[b])

        def wait_w(c):
            b = c % NTOT
            dst = out_hbm.at[pl.ds(base + c * CH, CH)]
            pltpu.make_async_copy(rows_v.at[b], dst, osem.at[b]).wait()

        for g in range(K):
            start_g(g)

        def body_early(g, carry):
            start_g(g + K)
            wait_g(g)
            start_w(g)
            return carry

        lax.fori_loop(0, NTOT - K, body_early, 0)

        def body_main(g, carry):
            wait_w(g + K - NTOT)
            start_g(g + K)
            wait_g(g)
            start_w(g)
            return carry

        lax.fori_loop(NTOT - K, nch - K, body_main, 0)

        def body_tail(g, carry):
            wait_g(g)
            start_w(g)
            return carry

        lax.fori_loop(nch - K, nch, body_tail, 0)

        def body_drain(g, carry):
            wait_w(g)
            return carry

        lax.fori_loop(nch - NTOT, nch, body_drain, 0)

    return k(table, idx)


def kernel(nodes_list, id2emb):
    batch, hist = nodes_list.shape
    B = batch * hist
    assert B % (NW * CH) == 0
    nch = B // (NW * CH)
    idx = nodes_list.astype(jnp.int32).reshape(NW, nch, CH)
    out = _gather(nch, idx, id2emb)
    return out.reshape(batch, hist, D)
